# baseline (speedup 1.0000x reference)
import jax
import jax.numpy as jnp
from jax import lax
from jax.experimental import pallas as pl
from jax.experimental.pallas import tpu as pltpu, tpu_sc as plsc

_B = 16384
_DIM = 32
_INFO = plsc.get_sparse_core_info()
_NC = _INFO.num_cores
_NS = _INFO.num_subcores
_NW = _NC * _NS
_BPW = _B // _NW
_CHUNK = 128
_NCHUNK = _BPW // _CHUNK


def _mf_body(user_ref, item_ref, ue_ref, ie_ref, ub_ref, ib_ref, gb_ref,
             out_ref,
             idx_u, idx_i, rows_u, rows_i, bu, bi, gb_v, out_v, sem):
    wid = lax.axis_index("s") * _NC + lax.axis_index("c")
    base = wid * _BPW

    pltpu.sync_copy(user_ref.at[wid], idx_u)
    pltpu.sync_copy(item_ref.at[wid], idx_i)
    pltpu.sync_copy(gb_ref, gb_v)

    copies = []
    for j in range(_NCHUNK):
        copies.append(pltpu.async_copy(ue_ref.at[idx_u.at[j]], rows_u.at[j], sem))
        copies.append(pltpu.async_copy(ie_ref.at[idx_i.at[j]], rows_i.at[j], sem))
        copies.append(pltpu.async_copy(ub_ref.at[idx_u.at[j]], bu.at[j], sem))
        copies.append(pltpu.async_copy(ib_ref.at[idx_i.at[j]], bi.at[j], sem))
    for c in copies:
        c.wait()

    gb = gb_v[...]
    lane = lax.iota(jnp.int32, 16)

    for j in range(_NCHUNK):
        def body(g, carry, j=j):
            acc = jnp.zeros((16,), jnp.float32)
            for k in range(16):
                r = g * 16 + k
                ua = rows_u[j, r, pl.ds(0, 16)]
                ub2 = rows_u[j, r, pl.ds(16, 16)]
                ia = rows_i[j, r, pl.ds(0, 16)]
                ib2 = rows_i[j, r, pl.ds(16, 16)]
                t = ua * ia + ub2 * ib2
                s = jnp.sum(t)
                acc = jnp.where(lane == k, s, acc)
            bvec = bu[j, pl.ds(g * 16, 16)] + bi[j, pl.ds(g * 16, 16)]
            out_v[pl.ds(j * _CHUNK + g * 16, 16)] = acc + bvec + gb
            return carry
        lax.fori_loop(0, _CHUNK // 16, body, 0)

    pltpu.sync_copy(out_v, out_ref.at[pl.ds(base, _BPW)])


def kernel(user, item, user_emb, item_emb, user_bias, item_bias, global_bias):
    user = user.astype(jnp.int32).reshape(_NW, _NCHUNK, _CHUNK)
    item = item.astype(jnp.int32).reshape(_NW, _NCHUNK, _CHUNK)
    user_bias = user_bias.reshape(-1)
    item_bias = item_bias.reshape(-1)
    gb = jnp.broadcast_to(global_bias.astype(jnp.float32), (16,))

    mesh = plsc.VectorSubcoreMesh(core_axis_name="c", subcore_axis_name="s")
    f = pl.kernel(
        _mf_body,
        out_type=jax.ShapeDtypeStruct((_B,), jnp.float32),
        mesh=mesh,
        compiler_params=pltpu.CompilerParams(needs_layout_passes=False,
                                             use_tc_tiling_on_sc=False),
        scratch_types=[
            pltpu.VMEM((_NCHUNK, _CHUNK), jnp.int32),
            pltpu.VMEM((_NCHUNK, _CHUNK), jnp.int32),
            pltpu.VMEM((_NCHUNK, _CHUNK, _DIM), jnp.float32),
            pltpu.VMEM((_NCHUNK, _CHUNK, _DIM), jnp.float32),
            pltpu.VMEM((_NCHUNK, _CHUNK), jnp.float32),
            pltpu.VMEM((_NCHUNK, _CHUNK), jnp.float32),
            pltpu.VMEM((16,), jnp.float32),
            pltpu.VMEM((_BPW,), jnp.float32),
            pltpu.SemaphoreType.DMA,
        ],
    )
    return f(user, item, user_emb, item_emb, user_bias, item_bias, gb)
